# baseline (speedup 1.0000x reference)
import functools

import jax
import jax.numpy as jnp
from jax import lax
from jax.experimental import pallas as pl
from jax.experimental.pallas import tpu as pltpu
from jax.experimental.pallas import tpu_sc as plsc

_B = 4
_S = 2048
_D = 768
_EPS = 1e-12

_L = 16
_NSL = _D // _L
_N_TOK = _B * _S
_NW = 32
_TPW = _N_TOK // _NW
_K = 32
_NCHUNK = _TPW // _K


def _body(ids_hbm, wemb_hbm, pemb_hbm, temb_hbm, gam_hbm, bet_hbm, out_hbm,
          idx_v, row_v, pos_v, typ_v, gam_v, bet_v, sem):
    cid = lax.axis_index("c")
    sid = lax.axis_index("s")
    wid = sid * 2 + cid
    base = wid * _TPW

    pltpu.sync_copy(temb_hbm.at[0], typ_v)
    pltpu.sync_copy(gam_hbm, gam_v)
    pltpu.sync_copy(bet_hbm, bet_v)

    def chunk(c, carry):
        t0 = base + c * _K
        p0 = lax.rem(t0, _S)
        pltpu.sync_copy(ids_hbm.at[pl.ds(t0, _K)], idx_v)
        gather = pltpu.async_copy(wemb_hbm.at[idx_v], row_v, sem)
        pltpu.sync_copy(pemb_hbm.at[pl.ds(p0, _K)], pos_v)
        gather.wait()

        def token(i, carry2):
            zero = jnp.zeros((_L,), jnp.float32)

            def red(j, acc):
                s, s2 = acc
                sl = pl.ds(j * _L, _L)
                x = row_v[i, sl] + pos_v[i, sl] + typ_v[sl]
                row_v[i, sl] = x
                return (s + x, s2 + x * x)

            s, s2 = lax.fori_loop(0, _NSL, red, (zero, zero))
            mean = jnp.sum(s) * (1.0 / _D)
            var = jnp.sum(s2) * (1.0 / _D) - mean * mean + _EPS
            vv = jnp.full((_L,), var, jnp.float32)
            ii = lax.bitcast_convert_type(vv, jnp.int32)
            ii = jnp.int32(0x5F3759DF) - lax.shift_right_logical(ii, 1)
            y = lax.bitcast_convert_type(ii, jnp.float32)
            for _ in range(3):
                y = y * (1.5 - 0.5 * vv * y * y)
            mv = jnp.full((_L,), mean, jnp.float32)

            def wr(j, _):
                sl = pl.ds(j * _L, _L)
                x = row_v[i, sl]
                row_v[i, sl] = (x - mv) * y * gam_v[sl] + bet_v[sl]
                return 0

            lax.fori_loop(0, _NSL, wr, 0)
            return carry2

        lax.fori_loop(0, _K, token, 0)
        pltpu.sync_copy(row_v, out_hbm.at[pl.ds(t0, _K)])
        return carry

    lax.fori_loop(0, _NCHUNK, chunk, 0)


@jax.jit
def _run(ids, wemb, pemb, temb, gam, bet):
    mesh = plsc.VectorSubcoreMesh(core_axis_name="c", subcore_axis_name="s")
    f = functools.partial(
        pl.kernel,
        mesh=mesh,
        compiler_params=pltpu.CompilerParams(needs_layout_passes=False),
        out_type=jax.ShapeDtypeStruct((_N_TOK, _D), jnp.float32),
        scratch_types=[
            pltpu.VMEM((_K,), jnp.int32),
            pltpu.VMEM((_K, _D), jnp.float32),
            pltpu.VMEM((_K, _D), jnp.float32),
            pltpu.VMEM((_D,), jnp.float32),
            pltpu.VMEM((_D,), jnp.float32),
            pltpu.VMEM((_D,), jnp.float32),
            pltpu.SemaphoreType.DMA,
        ],
    )(_body)
    return f(ids, wemb, pemb, temb, gam, bet)


def kernel(input_ids, word_emb, pos_emb, type_emb, gamma, beta):
    b, s = input_ids.shape
    ids = input_ids.reshape(-1).astype(jnp.int32)
    out = _run(ids, word_emb, pos_emb, type_emb, gamma, beta)
    return out.reshape(b, s, _D).astype(jnp.bfloat16)


# speedup vs baseline: 4.7166x; 4.7166x over previous
import functools

import jax
import jax.numpy as jnp
from jax import lax
from jax.experimental import pallas as pl
from jax.experimental.pallas import tpu as pltpu
from jax.experimental.pallas import tpu_sc as plsc

_B = 4
_S = 2048
_D = 768
_EPS = 1e-12

_N_TOK = _B * _S
_NW = 32
_TPW = _N_TOK // _NW
_K = 64
_NCH = _TPW // _K

_BLK_T = 1024


def _gather_body(ids_hbm, wemb_hbm, out_hbm,
                 idx0, idx1, row0, row1, sg0, sg1, ss0, ss1):
    cid = lax.axis_index("c")
    sid = lax.axis_index("s")
    base = (sid * 2 + cid) * _TPW
    idx = (idx0, idx1)
    row = (row0, row1)
    sg = (sg0, sg1)
    ss = (ss0, ss1)

    pltpu.sync_copy(ids_hbm.at[pl.ds(base, _K)], idx0)
    pltpu.async_copy(wemb_hbm.at[idx0], row0, sg0)
    for c in range(_NCH):
        b = c & 1
        if c + 1 < _NCH:
            pltpu.sync_copy(ids_hbm.at[pl.ds(base + (c + 1) * _K, _K)],
                            idx[1 - b])
            if c >= 1:
                pltpu.make_async_copy(
                    row[1 - b], out_hbm.at[pl.ds(base + (c - 1) * _K, _K)],
                    ss[1 - b]).wait()
            pltpu.async_copy(wemb_hbm.at[idx[1 - b]], row[1 - b], sg[1 - b])
        pltpu.make_async_copy(wemb_hbm.at[idx[b]], row[b], sg[b]).wait()
        pltpu.async_copy(row[b], out_hbm.at[pl.ds(base + c * _K, _K)], ss[b])
    for c in (_NCH - 2, _NCH - 1):
        b = c & 1
        pltpu.make_async_copy(
            row[b], out_hbm.at[pl.ds(base + c * _K, _K)], ss[b]).wait()


def _sc_gather(ids, wemb):
    mesh = plsc.VectorSubcoreMesh(core_axis_name="c", subcore_axis_name="s")
    f = functools.partial(
        pl.kernel,
        mesh=mesh,
        compiler_params=pltpu.CompilerParams(needs_layout_passes=False),
        out_type=jax.ShapeDtypeStruct((_N_TOK, _D), jnp.float32),
        scratch_types=[
            pltpu.VMEM((_K,), jnp.int32),
            pltpu.VMEM((_K,), jnp.int32),
            pltpu.VMEM((_K, _D), jnp.float32),
            pltpu.VMEM((_K, _D), jnp.float32),
            pltpu.SemaphoreType.DMA,
            pltpu.SemaphoreType.DMA,
            pltpu.SemaphoreType.DMA,
            pltpu.SemaphoreType.DMA,
        ],
    )(_gather_body)
    return f(ids, wemb)


def _ln_body(rows_ref, pos_ref, typ_ref, gam_ref, bet_ref, out_ref):
    x = rows_ref[...] + pos_ref[...] + typ_ref[...]
    mean = jnp.mean(x, axis=1, keepdims=True)
    xc = x - mean
    var = jnp.mean(xc * xc, axis=1, keepdims=True)
    y = xc * lax.rsqrt(var + _EPS)
    out_ref[...] = (y * gam_ref[...] + bet_ref[...]).astype(jnp.bfloat16)


def _tc_layernorm(rows, pos, typ0, gam2, bet2):
    nh = _S // _BLK_T
    return pl.pallas_call(
        _ln_body,
        grid=(nh, _B),
        in_specs=[
            pl.BlockSpec((_BLK_T, _D), lambda h, b: (b * nh + h, 0)),
            pl.BlockSpec((_BLK_T, _D), lambda h, b: (h, 0)),
            pl.BlockSpec((1, _D), lambda h, b: (0, 0)),
            pl.BlockSpec((1, _D), lambda h, b: (0, 0)),
            pl.BlockSpec((1, _D), lambda h, b: (0, 0)),
        ],
        out_specs=pl.BlockSpec((_BLK_T, _D), lambda h, b: (b * nh + h, 0)),
        out_shape=jax.ShapeDtypeStruct((_N_TOK, _D), jnp.bfloat16),
    )(rows, pos, typ0, gam2, bet2)


@jax.jit
def _run(ids, wemb, pemb, temb, gam, bet):
    rows = _sc_gather(ids, wemb)
    typ0 = temb[0:1]
    return _tc_layernorm(rows, pemb[:_S], typ0, gam.reshape(1, _D),
                         bet.reshape(1, _D))


def kernel(input_ids, word_emb, pos_emb, type_emb, gamma, beta):
    b, s = input_ids.shape
    ids = input_ids.reshape(-1).astype(jnp.int32)
    out = _run(ids, word_emb, pos_emb, type_emb, gamma, beta)
    return out.reshape(b, s, _D)
